# baseline (speedup 1.0000x reference)
import jax
import jax.numpy as jnp
from jax import lax
from jax.experimental import pallas as pl
from jax.experimental.pallas import tpu as pltpu
from jax.experimental.pallas import tpu_sc as plsc

B = 16384
A = 1000
NC = 2
NS = 16
NW = NC * NS
RPW = B // NW
CR = 32
NCH = RPW // CR
NBUF = 3


def _sc_body(sav_hbm, act_hbm, q_hbm, out_hbm, act_v, q_v, bufs, lsems, ssems):
    wid = lax.axis_index("s") * NC + lax.axis_index("c")
    base = wid * RPW

    pltpu.sync_copy(act_hbm.at[pl.ds(base, RPW)], act_v)
    pltpu.sync_copy(q_hbm.at[pl.ds(base, RPW)], q_v)

    def start_load(g):
        b = g % NBUF
        return pltpu.async_copy(
            sav_hbm.at[pl.ds(base + g * CR, CR), :], bufs.at[b], lsems[b])

    def start_store(g):
        b = g % NBUF
        return pltpu.async_copy(
            bufs.at[b], out_hbm.at[pl.ds(base + g * CR, CR), :], ssems[b])

    loads = {}
    stores = {}
    for g in range(min(NBUF - 1, NCH)):
        loads[g] = start_load(g)

    rows = lax.iota(jnp.int32, 16)
    for g in range(NCH):
        b = g % NBUF
        loads.pop(g).wait()
        for j in range(CR // 16):
            off = g * CR + j * 16
            cols = act_v[pl.ds(off, 16)]
            vals = q_v[pl.ds(off, 16)]
            plsc.store_scatter(bufs.at[b], [rows + j * 16, cols], vals)
        stores[g] = start_store(g)
        nxt = g + NBUF - 1
        if nxt < NCH:
            if nxt >= NBUF:
                stores.pop(nxt - NBUF).wait()
            loads[nxt] = start_load(nxt)
    for g in sorted(stores):
        stores[g].wait()


def kernel(state_action_values, action, q_prime):
    act = action[:, 0].astype(jnp.int32)
    mesh = plsc.VectorSubcoreMesh(
        core_axis_name="c", subcore_axis_name="s", num_cores=NC,
        num_subcores=NS)
    sc_call = pl.kernel(
        _sc_body,
        out_type=jax.ShapeDtypeStruct((B, A), jnp.float32),
        mesh=mesh,
        compiler_params=pltpu.CompilerParams(
            needs_layout_passes=False,
            use_tc_tiling_on_sc=True,
            disable_bounds_checks=True,
        ),
        scratch_types=[
            pltpu.VMEM((RPW,), jnp.int32),
            pltpu.VMEM((RPW,), jnp.float32),
            pltpu.VMEM((NBUF, CR, A), jnp.float32),
            [pltpu.SemaphoreType.DMA] * NBUF,
            [pltpu.SemaphoreType.DMA] * NBUF,
        ],
    )
    return sc_call(state_action_values, act, q_prime)


# speedup vs baseline: 2.4363x; 2.4363x over previous
import jax
import jax.numpy as jnp
from jax import lax
from jax.experimental import pallas as pl
from jax.experimental.pallas import tpu as pltpu
from jax.experimental.pallas import tpu_sc as plsc

B = 16384
A = 1000
NC = 2
NS = 16
NW = NC * NS
LPW = B // NW
RT = 40
NCH = A // RT
NBUF = 4


def _sc_body(sav_hbm, act_hbm, q_hbm, out_hbm, act_v, q_v, bufs, lsems, ssems):
    wid = lax.axis_index("s") * NC + lax.axis_index("c")
    base = wid * LPW

    pltpu.sync_copy(act_hbm.at[pl.ds(base, LPW)], act_v)
    pltpu.sync_copy(q_hbm.at[pl.ds(base, LPW)], q_v)

    def start_load(g):
        b = g % NBUF
        return pltpu.async_copy(
            sav_hbm.at[pl.ds(g * RT, RT), pl.ds(base, LPW)], bufs.at[b],
            lsems[b])

    def start_store(g):
        b = g % NBUF
        return pltpu.async_copy(
            bufs.at[b], out_hbm.at[pl.ds(g * RT, RT), pl.ds(base, LPW)],
            ssems[b])

    loads = {}
    stores = {}
    for g in range(min(NBUF - 1, NCH)):
        loads[g] = start_load(g)

    lane = lax.iota(jnp.int32, 16)
    for g in range(NCH):
        b = g % NBUF
        loads.pop(g).wait()
        r0 = g * RT
        for k in range(LPW // 16):
            cols = act_v[pl.ds(k * 16, 16)]
            vals = q_v[pl.ds(k * 16, 16)]
            mask = (cols >= r0) & (cols < r0 + RT)
            plsc.store_scatter(
                bufs.at[b], [cols - r0, lane + k * 16], vals, mask=mask)
        stores[g] = start_store(g)
        nxt = g + NBUF - 1
        if nxt < NCH:
            if nxt >= NBUF:
                stores.pop(nxt - NBUF).wait()
            loads[nxt] = start_load(nxt)
    for g in sorted(stores):
        stores[g].wait()


def kernel(state_action_values, action, q_prime):
    act = action[:, 0].astype(jnp.int32)
    sav_t = state_action_values.T
    mesh = plsc.VectorSubcoreMesh(
        core_axis_name="c", subcore_axis_name="s", num_cores=NC,
        num_subcores=NS)
    sc_call = pl.kernel(
        _sc_body,
        out_type=jax.ShapeDtypeStruct((A, B), jnp.float32),
        mesh=mesh,
        compiler_params=pltpu.CompilerParams(needs_layout_passes=False),
        scratch_types=[
            pltpu.VMEM((LPW,), jnp.int32),
            pltpu.VMEM((LPW,), jnp.float32),
            pltpu.VMEM((NBUF, RT, LPW), jnp.float32),
            [pltpu.SemaphoreType.DMA] * NBUF,
            [pltpu.SemaphoreType.DMA] * NBUF,
        ],
    )
    return sc_call(sav_t, act, q_prime).T
